# baseline (speedup 1.0000x reference)
import functools

import jax
import jax.numpy as jnp
from jax import lax
from jax.experimental import pallas as pl
from jax.experimental.pallas import tpu as pltpu
from jax.experimental.pallas import tpu_sc as plsc

BATCH = 16384
HIST = 20
EMBED_DIM = 32
N = BATCH * HIST

NUM_CORES = 2
NUM_SUBCORES = 16
NUM_WORKERS = NUM_CORES * NUM_SUBCORES
BATCH_PER_WORKER = BATCH // NUM_WORKERS
CB = 64
ROWS = CB * HIST
NUM_CHUNKS = BATCH_PER_WORKER // CB
LANES = 16


def _sc_fused(table_hbm, idx_hbm, x_hbm, out_hbm, idx_v, rows_v, x_v,
              sem_g, sem_x):
    wid = lax.axis_index("s") * NUM_CORES + lax.axis_index("c")
    base = wid * BATCH_PER_WORKER

    @pl.loop(0, NUM_CHUNKS)
    def _(ci):
        b0 = base + ci * CB
        pltpu.sync_copy(idx_hbm.at[pl.ds(b0 * HIST, ROWS)], idx_v)
        cg = pltpu.async_copy(table_hbm.at[idx_v], rows_v, sem_g)
        cx = pltpu.async_copy(x_hbm.at[pl.ds(b0, CB)], x_v, sem_x)
        cx.wait()
        cg.wait()

        @pl.loop(0, CB)
        def _(b):
            @pl.loop(0, HIST)
            def _(h):
                r = b * HIST + h
                for c in range(0, EMBED_DIM, LANES):
                    xv = x_v[b, h, pl.ds(c, LANES)]
                    g = rows_v[r, pl.ds(c, LANES)]
                    x_v[b, h, pl.ds(c, LANES)] = g / (1.0 + jnp.exp(-2.0 * xv))

        pltpu.sync_copy(x_v, out_hbm.at[pl.ds(b0, CB)])


def kernel(x, index, supports_01):
    idx = index.reshape(N).astype(jnp.int32)

    fused = functools.partial(
        pl.kernel,
        out_type=jax.ShapeDtypeStruct((BATCH, HIST, EMBED_DIM), jnp.float32),
        mesh=plsc.VectorSubcoreMesh(core_axis_name="c", subcore_axis_name="s"),
        scratch_types=[
            pltpu.VMEM((ROWS,), jnp.int32),
            pltpu.VMEM((ROWS, EMBED_DIM), jnp.float32),
            pltpu.VMEM((CB, HIST, EMBED_DIM), jnp.float32),
            pltpu.SemaphoreType.DMA,
            pltpu.SemaphoreType.DMA,
        ],
        compiler_params=pltpu.CompilerParams(use_tc_tiling_on_sc=False),
    )(_sc_fused)

    return fused(supports_01, idx, x)


# speedup vs baseline: 1.1084x; 1.1084x over previous
import functools

import jax
import jax.numpy as jnp
from jax import lax
from jax.experimental import pallas as pl
from jax.experimental.pallas import tpu as pltpu
from jax.experimental.pallas import tpu_sc as plsc

BATCH = 16384
HIST = 20
EMBED_DIM = 32
N = BATCH * HIST
VOCAB = 1000000
PACK = 128 // EMBED_DIM

NUM_CORES = 2
NUM_SUBCORES = 16
NUM_WORKERS = NUM_CORES * NUM_SUBCORES
BATCH_PER_WORKER = BATCH // NUM_WORKERS
CB = 32
ROWS = CB * HIST
NUM_CHUNKS = BATCH_PER_WORKER // CB
LANES = 16

VPAD = 1024000
TBLK = 8192
STRIP = TBLK // PACK


def _tc_transpose_body(in_ref, out_ref):
    t = in_ref[...]
    out_ref[...] = jnp.concatenate(
        [t[:, q * STRIP:(q + 1) * STRIP].T for q in range(PACK)], axis=1)


def _pack_table(table_t):
    tp = jnp.pad(table_t, ((0, 0), (0, VPAD - VOCAB)))
    return pl.pallas_call(
        _tc_transpose_body,
        grid=(VPAD // TBLK,),
        in_specs=[pl.BlockSpec((EMBED_DIM, TBLK), lambda i: (0, i))],
        out_specs=pl.BlockSpec((STRIP, 128), lambda i: (i, 0)),
        out_shape=jax.ShapeDtypeStruct((VPAD // PACK, 128), jnp.float32),
    )(tp)


def _sc_fused(table_hbm, idx_hbm, x_hbm, out_hbm, idx_v, idx4_v, g_v, xo_v,
              sem_g, sem_x):
    wid = lax.axis_index("s") * NUM_CORES + lax.axis_index("c")
    base = wid * BATCH_PER_WORKER

    @pl.loop(0, NUM_CHUNKS)
    def _(ci):
        b0 = base + ci * CB
        n0 = b0 * HIST
        pltpu.sync_copy(idx_hbm.at[pl.ds(n0, ROWS)], idx_v)

        @pl.loop(0, ROWS, step=LANES)
        def _(i):
            v = idx_v[pl.ds(i, LANES)]
            idx4_v[pl.ds(i, LANES)] = (
                jax.lax.shift_left(jax.lax.shift_right_logical(v, 13), 11)
                | (v & 2047))

        cg = pltpu.async_copy(table_hbm.at[idx4_v], g_v, sem_g)
        cx = pltpu.async_copy(x_hbm.at[:, :, pl.ds(b0, CB)], xo_v, sem_x)
        cx.wait()
        cg.wait()

        @pl.loop(0, CB // LANES)
        def _(bg):
            lane_b = lax.broadcasted_iota(jnp.int32, (LANES,), 0)
            for h in range(HIST):
                rvec = (bg * LANES + lane_b) * HIST + h
                iv = plsc.load_gather(idx_v, [rvec])
                c0v = (jax.lax.shift_right_logical(iv, 11)
                       & (PACK - 1)) * EMBED_DIM
                for d in range(EMBED_DIM):
                    g = plsc.load_gather(g_v, [rvec, c0v + d])
                    xv = xo_v[h, d, pl.ds(bg * LANES, LANES)]
                    xo_v[h, d, pl.ds(bg * LANES, LANES)] = (
                        g / (1.0 + jnp.exp(-2.0 * xv)))

        pltpu.sync_copy(xo_v, out_hbm.at[:, :, pl.ds(b0, CB)])


def kernel(x, index, supports_01):
    xp = x.transpose(1, 2, 0)
    tt = _pack_table(supports_01.T)
    idx = index.reshape(N).astype(jnp.int32)

    fused = functools.partial(
        pl.kernel,
        out_type=jax.ShapeDtypeStruct((HIST, EMBED_DIM, BATCH), jnp.float32),
        mesh=plsc.VectorSubcoreMesh(core_axis_name="c", subcore_axis_name="s"),
        scratch_types=[
            pltpu.VMEM((ROWS,), jnp.int32),
            pltpu.VMEM((ROWS,), jnp.int32),
            pltpu.VMEM((ROWS, 128), jnp.float32),
            pltpu.VMEM((HIST, EMBED_DIM, CB), jnp.float32),
            pltpu.SemaphoreType.DMA,
            pltpu.SemaphoreType.DMA,
        ],
        compiler_params=pltpu.CompilerParams(use_tc_tiling_on_sc=False,
                                             needs_layout_passes=False),
    )(_sc_fused)

    out_t = fused(tt, idx, xp)
    return out_t.transpose(2, 0, 1)


# speedup vs baseline: 1.3459x; 1.2143x over previous
import functools

import jax
import jax.numpy as jnp
from jax import lax
from jax.experimental import pallas as pl
from jax.experimental.pallas import tpu as pltpu
from jax.experimental.pallas import tpu_sc as plsc

BATCH = 16384
HIST = 20
EMBED_DIM = 32
N = BATCH * HIST
VOCAB = 1000000
PACK = 128 // EMBED_DIM

NUM_CORES = 2
NUM_SUBCORES = 16
NUM_WORKERS = NUM_CORES * NUM_SUBCORES
BATCH_PER_WORKER = BATCH // NUM_WORKERS
CB = 64
ROWS = CB * HIST
NUM_CHUNKS = BATCH_PER_WORKER // CB
LANES = 16

VPAD = 1024000
TBLK = 8192
STRIP = TBLK // PACK


def _tc_transpose_body(in_ref, out_ref):
    t = in_ref[...]
    out_ref[...] = jnp.concatenate(
        [t[:, q * STRIP:(q + 1) * STRIP].T for q in range(PACK)], axis=1)


def _pack_table(table_t):
    tp = jnp.pad(table_t, ((0, 0), (0, VPAD - VOCAB)))
    return pl.pallas_call(
        _tc_transpose_body,
        grid=(VPAD // TBLK,),
        in_specs=[pl.BlockSpec((EMBED_DIM, TBLK), lambda i: (0, i))],
        out_specs=pl.BlockSpec((STRIP, 128), lambda i: (i, 0)),
        out_shape=jax.ShapeDtypeStruct((VPAD // PACK, 128), jnp.float32),
    )(tp)


def _sc_fused(table_hbm, idx_hbm, x_hbm, out_hbm, idx_v, idx4_v, g_v, xo_v,
              sem_g, sem_x):
    wid = lax.axis_index("s") * NUM_CORES + lax.axis_index("c")
    base = wid * BATCH_PER_WORKER

    @pl.loop(0, NUM_CHUNKS)
    def _(ci):
        b0 = base + ci * CB
        n0 = b0 * HIST
        pltpu.sync_copy(idx_hbm.at[pl.ds(n0, ROWS)], idx_v)

        @pl.loop(0, ROWS, step=LANES)
        def _(i):
            v = idx_v[pl.ds(i, LANES)]
            shr = jax.lax.shift_right_logical
            shl = jax.lax.shift_left
            idx4_v[pl.ds(i, LANES)] = (
                shl(shr(v, 13), 13) | shl(v & 2047, 2) | (shr(v, 11) & 3))

        cg = pltpu.async_copy(table_hbm.at[idx4_v], g_v, sem_g)
        cx = pltpu.async_copy(x_hbm.at[:, :, pl.ds(b0, CB)], xo_v, sem_x)
        cx.wait()
        cg.wait()

        @pl.loop(0, CB // LANES)
        def _(bg):
            lane_r = lax.broadcasted_iota(jnp.int32, (LANES,), 0) * HIST
            for h in range(HIST):
                rvec = bg * (LANES * HIST) + lane_r + h
                for d in range(EMBED_DIM):
                    dvec = jnp.full((LANES,), d, jnp.int32)
                    g = plsc.load_gather(g_v, [rvec, dvec])
                    xv = xo_v[h, d, pl.ds(bg * LANES, LANES)]
                    xo_v[h, d, pl.ds(bg * LANES, LANES)] = (
                        g / (1.0 + jnp.exp(-2.0 * xv)))

        pltpu.sync_copy(xo_v, out_hbm.at[:, :, pl.ds(b0, CB)])


def kernel(x, index, supports_01):
    xp = x.transpose(1, 2, 0)
    tt = _pack_table(supports_01.T).reshape(VPAD * EMBED_DIM).reshape(
        VPAD, EMBED_DIM)
    idx = index.reshape(N).astype(jnp.int32)

    fused = functools.partial(
        pl.kernel,
        out_type=jax.ShapeDtypeStruct((HIST, EMBED_DIM, BATCH), jnp.float32),
        mesh=plsc.VectorSubcoreMesh(core_axis_name="c", subcore_axis_name="s"),
        scratch_types=[
            pltpu.VMEM((ROWS,), jnp.int32),
            pltpu.VMEM((ROWS,), jnp.int32),
            pltpu.VMEM((ROWS, EMBED_DIM), jnp.float32),
            pltpu.VMEM((HIST, EMBED_DIM, CB), jnp.float32),
            pltpu.SemaphoreType.DMA,
            pltpu.SemaphoreType.DMA,
        ],
        compiler_params=pltpu.CompilerParams(use_tc_tiling_on_sc=False,
                                             needs_layout_passes=False),
    )(_sc_fused)

    out_t = fused(tt, idx, xp)
    return out_t.transpose(2, 0, 1)


# speedup vs baseline: 1.8270x; 1.3574x over previous
import functools

import jax
import jax.numpy as jnp
from jax import lax
from jax.experimental import pallas as pl
from jax.experimental.pallas import tpu as pltpu
from jax.experimental.pallas import tpu_sc as plsc

BATCH = 16384
HIST = 20
EMBED_DIM = 32
N = BATCH * HIST
VOCAB = 1000000
PACK = 128 // EMBED_DIM

NUM_CORES = 2
NUM_SUBCORES = 16
NUM_WORKERS = NUM_CORES * NUM_SUBCORES
BATCH_PER_WORKER = BATCH // NUM_WORKERS
CB = 64
ROWS = CB * HIST
NUM_CHUNKS = BATCH_PER_WORKER // CB
LANES = 16

VPAD = 1024000
TBLK = 8192
STRIP = TBLK // PACK


def _tc_transpose_body(in_ref, out_ref):
    t = in_ref[...]
    stacked = jnp.concatenate(
        [t[:, q * STRIP:(q + 1) * STRIP] for q in range(PACK)], axis=0)
    out_ref[...] = stacked.T


def _pack_table(table_t):
    tp = jnp.pad(table_t, ((0, 0), (0, VPAD - VOCAB)))
    return pl.pallas_call(
        _tc_transpose_body,
        grid=(VPAD // TBLK,),
        in_specs=[pl.BlockSpec((EMBED_DIM, TBLK), lambda i: (0, i))],
        out_specs=pl.BlockSpec((STRIP, 128), lambda i: (i, 0)),
        out_shape=jax.ShapeDtypeStruct((VPAD // PACK, 128), jnp.float32),
    )(tp)


SBLK = 2048


def _tc_sigmoid_body(x_ref, o_ref):
    xv = x_ref[...]
    o_ref[...] = 1.0 / (1.0 + jnp.exp(-2.0 * xv))


def _sigmoid_plane(xp):
    return pl.pallas_call(
        _tc_sigmoid_body,
        grid=(BATCH // SBLK,),
        in_specs=[pl.BlockSpec((HIST, EMBED_DIM, SBLK), lambda i: (0, 0, i))],
        out_specs=pl.BlockSpec((HIST, EMBED_DIM, SBLK), lambda i: (0, 0, i)),
        out_shape=jax.ShapeDtypeStruct((HIST, EMBED_DIM, BATCH), jnp.float32),
    )(xp)


def _sc_fused(table_hbm, idx_hbm, x_hbm, out_hbm, idx_v, idx4_v, g_v, xo_v,
              sem_g, sem_x):
    wid = lax.axis_index("s") * NUM_CORES + lax.axis_index("c")
    base = wid * BATCH_PER_WORKER

    @pl.loop(0, NUM_CHUNKS)
    def _(ci):
        b0 = base + ci * CB
        n0 = b0 * HIST
        pltpu.sync_copy(idx_hbm.at[pl.ds(n0, ROWS)], idx_v)

        @pl.loop(0, ROWS, step=LANES)
        def _(i):
            v = idx_v[pl.ds(i, LANES)]
            shr = jax.lax.shift_right_logical
            shl = jax.lax.shift_left
            idx4_v[pl.ds(i, LANES)] = (
                shl(shr(v, 13), 13) | shl(v & 2047, 2) | (shr(v, 11) & 3))

        cg = pltpu.async_copy(table_hbm.at[idx4_v], g_v, sem_g)
        cx = pltpu.async_copy(x_hbm.at[:, :, pl.ds(b0, CB)], xo_v, sem_x)
        cx.wait()
        cg.wait()

        @pl.loop(0, CB // LANES)
        def _(bg):
            lane_r = lax.broadcasted_iota(jnp.int32, (LANES,), 0) * HIST
            for h in range(HIST):
                rvec = bg * (LANES * HIST) + lane_r + h
                for d in range(EMBED_DIM):
                    dvec = jnp.full((LANES,), d, jnp.int32)
                    g = plsc.load_gather(g_v, [rvec, dvec])
                    sv = xo_v[h, d, pl.ds(bg * LANES, LANES)]
                    xo_v[h, d, pl.ds(bg * LANES, LANES)] = g * sv

        pltpu.sync_copy(xo_v, out_hbm.at[:, :, pl.ds(b0, CB)])


def kernel(x, index, supports_01):
    xp = _sigmoid_plane(x.transpose(1, 2, 0))
    tt = _pack_table(supports_01.T).reshape(VPAD * EMBED_DIM).reshape(
        VPAD, EMBED_DIM)
    idx = index.reshape(N).astype(jnp.int32)

    fused = functools.partial(
        pl.kernel,
        out_type=jax.ShapeDtypeStruct((HIST, EMBED_DIM, BATCH), jnp.float32),
        mesh=plsc.VectorSubcoreMesh(core_axis_name="c", subcore_axis_name="s"),
        scratch_types=[
            pltpu.VMEM((ROWS,), jnp.int32),
            pltpu.VMEM((ROWS,), jnp.int32),
            pltpu.VMEM((ROWS, EMBED_DIM), jnp.float32),
            pltpu.VMEM((HIST, EMBED_DIM, CB), jnp.float32),
            pltpu.SemaphoreType.DMA,
            pltpu.SemaphoreType.DMA,
        ],
        compiler_params=pltpu.CompilerParams(use_tc_tiling_on_sc=False,
                                             needs_layout_passes=False),
    )(_sc_fused)

    out_t = fused(tt, idx, xp)
    return out_t.transpose(2, 0, 1)


# speedup vs baseline: 2.1000x; 1.1494x over previous
import functools

import jax
import jax.numpy as jnp
from jax import lax
from jax.experimental import pallas as pl
from jax.experimental.pallas import tpu as pltpu
from jax.experimental.pallas import tpu_sc as plsc

BATCH = 16384
HIST = 20
EMBED_DIM = 32
N = BATCH * HIST
VOCAB = 1000000
PACK = 128 // EMBED_DIM

NUM_CORES = 2
NUM_SUBCORES = 16
NUM_WORKERS = NUM_CORES * NUM_SUBCORES
BATCH_PER_WORKER = BATCH // NUM_WORKERS
CB = 64
ROWS = CB * HIST
NUM_CHUNKS = BATCH_PER_WORKER // CB
LANES = 16

VPAD = 1024000
TBLK = 8192
STRIP = TBLK // PACK


NBLK = 123
VTAIL = VOCAB - (NBLK - 1) * TBLK


def _tc_transpose_body(hbm_ref, tail_ref, out_ref, scr0, scr1, sem0, sem1):
    i = pl.program_id(0)

    def start(blk, scr, sem):
        @pl.when(blk < NBLK - 1)
        def _():
            pltpu.make_async_copy(
                hbm_ref.at[:, pl.ds(blk * TBLK, TBLK)], scr, sem).start()

        @pl.when(blk == NBLK - 1)
        def _():
            pltpu.make_async_copy(tail_ref, scr, sem).start()

    def wait(blk, scr, sem):
        @pl.when(blk < NBLK - 1)
        def _():
            pltpu.make_async_copy(
                hbm_ref.at[:, pl.ds(blk * TBLK, TBLK)], scr, sem).wait()

        @pl.when(blk == NBLK - 1)
        def _():
            pltpu.make_async_copy(tail_ref, scr, sem).wait()

    @pl.when(i == 0)
    def _():
        start(0, scr0, sem0)

    nxt = i + 1

    @pl.when(jnp.logical_and(nxt < NBLK, nxt % 2 == 0))
    def _():
        start(nxt, scr0, sem0)

    @pl.when(jnp.logical_and(nxt < NBLK, nxt % 2 == 1))
    def _():
        start(nxt, scr1, sem1)

    def emit(scr):
        t = scr[...]
        stacked = jnp.concatenate(
            [t[:, q * STRIP:(q + 1) * STRIP] for q in range(PACK)], axis=0)
        out_ref[...] = stacked.T

    @pl.when(i % 2 == 0)
    def _():
        wait(i, scr0, sem0)
        emit(scr0)

    @pl.when(i % 2 == 1)
    def _():
        wait(i, scr1, sem1)
        emit(scr1)


def _pack_table(table_t):
    tail = jnp.pad(
        lax.slice(table_t, (0, (NBLK - 1) * TBLK), (EMBED_DIM, VOCAB)),
        ((0, 0), (0, TBLK - VTAIL)))
    return pl.pallas_call(
        _tc_transpose_body,
        grid=(NBLK,),
        in_specs=[pl.BlockSpec(memory_space=pl.ANY),
                  pl.BlockSpec(memory_space=pl.ANY)],
        out_specs=pl.BlockSpec((STRIP, 128), lambda i: (i, 0)),
        out_shape=jax.ShapeDtypeStruct((VPAD // PACK, 128), jnp.float32),
        scratch_shapes=[
            pltpu.VMEM((EMBED_DIM, TBLK), jnp.float32),
            pltpu.VMEM((EMBED_DIM, TBLK), jnp.float32),
            pltpu.SemaphoreType.DMA,
            pltpu.SemaphoreType.DMA,
        ],
    )(table_t, tail)


SBLK = 2048


def _tc_sigmoid_body(x_ref, o_ref):
    xv = x_ref[...]
    o_ref[...] = 1.0 / (1.0 + jnp.exp(-2.0 * xv))


def _sigmoid_plane(xp):
    return pl.pallas_call(
        _tc_sigmoid_body,
        grid=(BATCH // SBLK,),
        in_specs=[pl.BlockSpec((HIST, EMBED_DIM, SBLK), lambda i: (0, 0, i))],
        out_specs=pl.BlockSpec((HIST, EMBED_DIM, SBLK), lambda i: (0, 0, i)),
        out_shape=jax.ShapeDtypeStruct((HIST, EMBED_DIM, BATCH), jnp.float32),
    )(xp)


def _sc_fused(table_hbm, idx_hbm, x_hbm, out_hbm, idx_v, idx4_v, g_v, xo_v,
              sem_g, sem_x):
    wid = lax.axis_index("s") * NUM_CORES + lax.axis_index("c")
    base = wid * BATCH_PER_WORKER

    @pl.loop(0, NUM_CHUNKS)
    def _(ci):
        b0 = base + ci * CB
        n0 = b0 * HIST
        pltpu.sync_copy(idx_hbm.at[pl.ds(n0, ROWS)], idx_v)

        @pl.loop(0, ROWS, step=LANES)
        def _(i):
            v = idx_v[pl.ds(i, LANES)]
            shr = jax.lax.shift_right_logical
            shl = jax.lax.shift_left
            idx4_v[pl.ds(i, LANES)] = (
                shl(shr(v, 13), 13) | shl(v & 2047, 2) | (shr(v, 11) & 3))

        cg = pltpu.async_copy(table_hbm.at[idx4_v], g_v, sem_g)
        cx = pltpu.async_copy(x_hbm.at[:, :, pl.ds(b0, CB)], xo_v, sem_x)
        cx.wait()
        cg.wait()

        @pl.loop(0, CB // LANES)
        def _(bg):
            lane_r = lax.broadcasted_iota(jnp.int32, (LANES,), 0) * HIST
            for h in range(HIST):
                rvec = bg * (LANES * HIST) + lane_r + h
                for d in range(EMBED_DIM):
                    dvec = jnp.full((LANES,), d, jnp.int32)
                    g = plsc.load_gather(g_v, [rvec, dvec])
                    sv = xo_v[h, d, pl.ds(bg * LANES, LANES)]
                    xo_v[h, d, pl.ds(bg * LANES, LANES)] = g * sv

        pltpu.sync_copy(xo_v, out_hbm.at[:, :, pl.ds(b0, CB)])


def kernel(x, index, supports_01):
    xp = _sigmoid_plane(x.transpose(1, 2, 0))
    tt = _pack_table(supports_01.T).reshape(VPAD * EMBED_DIM).reshape(
        VPAD, EMBED_DIM)
    idx = index.reshape(N).astype(jnp.int32)

    fused = functools.partial(
        pl.kernel,
        out_type=jax.ShapeDtypeStruct((HIST, EMBED_DIM, BATCH), jnp.float32),
        mesh=plsc.VectorSubcoreMesh(core_axis_name="c", subcore_axis_name="s"),
        scratch_types=[
            pltpu.VMEM((ROWS,), jnp.int32),
            pltpu.VMEM((ROWS,), jnp.int32),
            pltpu.VMEM((ROWS, EMBED_DIM), jnp.float32),
            pltpu.VMEM((HIST, EMBED_DIM, CB), jnp.float32),
            pltpu.SemaphoreType.DMA,
            pltpu.SemaphoreType.DMA,
        ],
        compiler_params=pltpu.CompilerParams(use_tc_tiling_on_sc=False,
                                             needs_layout_passes=False),
    )(_sc_fused)

    out_t = fused(tt, idx, xp)
    return out_t.transpose(2, 0, 1)


# speedup vs baseline: 3.9704x; 1.8906x over previous
import functools

import jax
import jax.numpy as jnp
from jax import lax
from jax.experimental import pallas as pl
from jax.experimental.pallas import tpu as pltpu
from jax.experimental.pallas import tpu_sc as plsc

BATCH = 16384
HIST = 20
EMBED_DIM = 32
N = BATCH * HIST
VOCAB = 1000000
PACK = 128 // EMBED_DIM

NUM_CORES = 2
NUM_SUBCORES = 16
NUM_WORKERS = NUM_CORES * NUM_SUBCORES
BATCH_PER_WORKER = BATCH // NUM_WORKERS
CB = 64
ROWS = CB * HIST
NUM_CHUNKS = BATCH_PER_WORKER // CB
LANES = 16

VPAD = 1024000
TBLK = 8192
STRIP = TBLK // PACK


NBLK = 123
VTAIL = VOCAB - (NBLK - 1) * TBLK


def _tc_transpose_body(hbm_ref, tail_ref, out_ref, scr0, scr1, sem0, sem1):
    i = pl.program_id(0)

    def start(blk, scr, sem):
        @pl.when(blk < NBLK - 1)
        def _():
            pltpu.make_async_copy(
                hbm_ref.at[:, pl.ds(blk * TBLK, TBLK)], scr, sem).start()

        @pl.when(blk == NBLK - 1)
        def _():
            pltpu.make_async_copy(tail_ref, scr, sem).start()

    def wait(blk, scr, sem):
        @pl.when(blk < NBLK - 1)
        def _():
            pltpu.make_async_copy(
                hbm_ref.at[:, pl.ds(blk * TBLK, TBLK)], scr, sem).wait()

        @pl.when(blk == NBLK - 1)
        def _():
            pltpu.make_async_copy(tail_ref, scr, sem).wait()

    @pl.when(i == 0)
    def _():
        start(0, scr0, sem0)

    nxt = i + 1

    @pl.when(jnp.logical_and(nxt < NBLK, nxt % 2 == 0))
    def _():
        start(nxt, scr0, sem0)

    @pl.when(jnp.logical_and(nxt < NBLK, nxt % 2 == 1))
    def _():
        start(nxt, scr1, sem1)

    def emit(scr):
        t = scr[...]
        stacked = jnp.concatenate(
            [t[:, q * STRIP:(q + 1) * STRIP] for q in range(PACK)], axis=0)
        out_ref[...] = stacked.T

    @pl.when(i % 2 == 0)
    def _():
        wait(i, scr0, sem0)
        emit(scr0)

    @pl.when(i % 2 == 1)
    def _():
        wait(i, scr1, sem1)
        emit(scr1)


def _pack_table(table_t):
    tail = jnp.pad(
        lax.slice(table_t, (0, (NBLK - 1) * TBLK), (EMBED_DIM, VOCAB)),
        ((0, 0), (0, TBLK - VTAIL)))
    return pl.pallas_call(
        _tc_transpose_body,
        grid=(NBLK,),
        in_specs=[pl.BlockSpec(memory_space=pl.ANY),
                  pl.BlockSpec(memory_space=pl.ANY)],
        out_specs=pl.BlockSpec((STRIP, 128), lambda i: (i, 0)),
        out_shape=jax.ShapeDtypeStruct((VPAD // PACK, 128), jnp.float32),
        scratch_shapes=[
            pltpu.VMEM((EMBED_DIM, TBLK), jnp.float32),
            pltpu.VMEM((EMBED_DIM, TBLK), jnp.float32),
            pltpu.SemaphoreType.DMA,
            pltpu.SemaphoreType.DMA,
        ],
    )(table_t, tail)


SBLK = 1024
GROWS = HIST * EMBED_DIM // 128


def _tc_mul_body(x_ref, g_ref, o_ref):
    gv = g_ref[...]
    g3 = gv.reshape(SBLK, GROWS, 128)
    planes = []
    for k in range(GROWS):
        rowk = g3[:, k, :]
        for m in range(4):
            planes.append(rowk[:, 32 * m:32 * (m + 1)].T)
    gp = jnp.stack(planes, axis=0)
    xv = x_ref[...]
    o_ref[...] = gp / (1.0 + jnp.exp(-2.0 * xv))


def _mul_plane(xp, g128):
    return pl.pallas_call(
        _tc_mul_body,
        grid=(BATCH // SBLK,),
        in_specs=[
            pl.BlockSpec((HIST, EMBED_DIM, SBLK), lambda i: (0, 0, i)),
            pl.BlockSpec((GROWS * SBLK, 128), lambda i: (i, 0)),
        ],
        out_specs=pl.BlockSpec((HIST, EMBED_DIM, SBLK), lambda i: (0, 0, i)),
        out_shape=jax.ShapeDtypeStruct((HIST, EMBED_DIM, BATCH), jnp.float32),
    )(xp, g128)


def _sc_gather(table_hbm, idx_hbm, out_hbm, idx_v, idx4_v, g_v, sem_g):
    wid = lax.axis_index("s") * NUM_CORES + lax.axis_index("c")
    base = wid * BATCH_PER_WORKER

    @pl.loop(0, NUM_CHUNKS)
    def _(ci):
        n0 = (base + ci * CB) * HIST
        pltpu.sync_copy(idx_hbm.at[pl.ds(n0, ROWS)], idx_v)

        @pl.loop(0, ROWS, step=LANES)
        def _(i):
            v = idx_v[pl.ds(i, LANES)]
            shr = jax.lax.shift_right_logical
            shl = jax.lax.shift_left
            idx4_v[pl.ds(i, LANES)] = (
                shl(shr(v, 13), 13) | shl(v & 2047, 2) | (shr(v, 11) & 3))

        pltpu.async_copy(table_hbm.at[idx4_v], g_v, sem_g).wait()
        pltpu.sync_copy(g_v, out_hbm.at[pl.ds(n0, ROWS)])


def kernel(x, index, supports_01):
    xp = x.transpose(1, 2, 0)
    tt = _pack_table(supports_01.T).reshape(VPAD * EMBED_DIM).reshape(
        VPAD, EMBED_DIM)
    idx = index.reshape(N).astype(jnp.int32)

    gather = functools.partial(
        pl.kernel,
        out_type=jax.ShapeDtypeStruct((N, EMBED_DIM), jnp.float32),
        mesh=plsc.VectorSubcoreMesh(core_axis_name="c", subcore_axis_name="s"),
        scratch_types=[
            pltpu.VMEM((ROWS,), jnp.int32),
            pltpu.VMEM((ROWS,), jnp.int32),
            pltpu.VMEM((ROWS, EMBED_DIM), jnp.float32),
            pltpu.SemaphoreType.DMA,
        ],
        compiler_params=pltpu.CompilerParams(use_tc_tiling_on_sc=False,
                                             needs_layout_passes=False),
    )(_sc_gather)

    g = gather(tt, idx)
    g128 = g.reshape(N * EMBED_DIM).reshape(N * EMBED_DIM // 128, 128)
    out_t = _mul_plane(xp, g128)
    return out_t.transpose(2, 0, 1)


# speedup vs baseline: 3.9776x; 1.0018x over previous
import functools

import jax
import jax.numpy as jnp
from jax import lax
from jax.experimental import pallas as pl
from jax.experimental.pallas import tpu as pltpu
from jax.experimental.pallas import tpu_sc as plsc

BATCH = 16384
HIST = 20
EMBED_DIM = 32
N = BATCH * HIST
VOCAB = 1000000
PACK = 128 // EMBED_DIM

NUM_CORES = 2
NUM_SUBCORES = 16
NUM_WORKERS = NUM_CORES * NUM_SUBCORES
BATCH_PER_WORKER = BATCH // NUM_WORKERS
CB = 64
ROWS = CB * HIST
NUM_CHUNKS = BATCH_PER_WORKER // CB
LANES = 16

VPAD = 1024000
TBLK = 8192
STRIP = TBLK // PACK


NBLK = 123
VTAIL = VOCAB - (NBLK - 1) * TBLK


def _tc_transpose_body(hbm_ref, tail_ref, out_ref, scr0, scr1, sem0, sem1):
    i = pl.program_id(0)

    def start(blk, scr, sem):
        @pl.when(blk < NBLK - 1)
        def _():
            pltpu.make_async_copy(
                hbm_ref.at[:, pl.ds(blk * TBLK, TBLK)], scr, sem).start()

        @pl.when(blk == NBLK - 1)
        def _():
            pltpu.make_async_copy(tail_ref, scr, sem).start()

    def wait(blk, scr, sem):
        @pl.when(blk < NBLK - 1)
        def _():
            pltpu.make_async_copy(
                hbm_ref.at[:, pl.ds(blk * TBLK, TBLK)], scr, sem).wait()

        @pl.when(blk == NBLK - 1)
        def _():
            pltpu.make_async_copy(tail_ref, scr, sem).wait()

    @pl.when(i == 0)
    def _():
        start(0, scr0, sem0)

    nxt = i + 1

    @pl.when(jnp.logical_and(nxt < NBLK, nxt % 2 == 0))
    def _():
        start(nxt, scr0, sem0)

    @pl.when(jnp.logical_and(nxt < NBLK, nxt % 2 == 1))
    def _():
        start(nxt, scr1, sem1)

    def emit(scr):
        t = scr[...]
        stacked = jnp.concatenate(
            [t[:, q * STRIP:(q + 1) * STRIP] for q in range(PACK)], axis=0)
        out_ref[...] = stacked.T

    @pl.when(i % 2 == 0)
    def _():
        wait(i, scr0, sem0)
        emit(scr0)

    @pl.when(i % 2 == 1)
    def _():
        wait(i, scr1, sem1)
        emit(scr1)


def _pack_table(table_t):
    tail = jnp.pad(
        lax.slice(table_t, (0, (NBLK - 1) * TBLK), (EMBED_DIM, VOCAB)),
        ((0, 0), (0, TBLK - VTAIL)))
    return pl.pallas_call(
        _tc_transpose_body,
        grid=(NBLK,),
        in_specs=[pl.BlockSpec(memory_space=pl.ANY),
                  pl.BlockSpec(memory_space=pl.ANY)],
        out_specs=pl.BlockSpec((STRIP, 128), lambda i: (i, 0)),
        out_shape=jax.ShapeDtypeStruct((VPAD // PACK, 128), jnp.float32),
        scratch_shapes=[
            pltpu.VMEM((EMBED_DIM, TBLK), jnp.float32),
            pltpu.VMEM((EMBED_DIM, TBLK), jnp.float32),
            pltpu.SemaphoreType.DMA,
            pltpu.SemaphoreType.DMA,
        ],
    )(table_t, tail)


SBLK = 1024
GROWS = HIST * EMBED_DIM // 128


def _tc_mul_body(x_ref, g_ref, o_ref):
    gv = g_ref[...]
    g3 = gv.reshape(SBLK, GROWS, 128)
    planes = []
    for k in range(GROWS):
        rowk = g3[:, k, :]
        for m in range(4):
            planes.append(rowk[:, 32 * m:32 * (m + 1)].T)
    gp = jnp.stack(planes, axis=0)
    xv = x_ref[...]
    o_ref[...] = gp / (1.0 + jnp.exp(-2.0 * xv))


def _mul_plane(xp, g128):
    return pl.pallas_call(
        _tc_mul_body,
        grid=(BATCH // SBLK,),
        in_specs=[
            pl.BlockSpec((HIST, EMBED_DIM, SBLK), lambda i: (0, 0, i)),
            pl.BlockSpec((GROWS * SBLK, 128), lambda i: (i, 0)),
        ],
        out_specs=pl.BlockSpec((HIST, EMBED_DIM, SBLK), lambda i: (0, 0, i)),
        out_shape=jax.ShapeDtypeStruct((HIST, EMBED_DIM, BATCH), jnp.float32),
        compiler_params=pltpu.CompilerParams(
            dimension_semantics=("parallel",)),
    )(xp, g128)


def _sc_gather(table_hbm, idx_hbm, out_hbm, idx_v, idx4_v, g_v, sem_g):
    wid = lax.axis_index("s") * NUM_CORES + lax.axis_index("c")
    base = wid * BATCH_PER_WORKER

    @pl.loop(0, NUM_CHUNKS)
    def _(ci):
        n0 = (base + ci * CB) * HIST
        pltpu.sync_copy(idx_hbm.at[pl.ds(n0, ROWS)], idx_v)

        @pl.loop(0, ROWS, step=LANES)
        def _(i):
            v = idx_v[pl.ds(i, LANES)]
            shr = jax.lax.shift_right_logical
            shl = jax.lax.shift_left
            idx4_v[pl.ds(i, LANES)] = (
                shl(shr(v, 13), 13) | shl(v & 2047, 2) | (shr(v, 11) & 3))

        pltpu.async_copy(table_hbm.at[idx4_v], g_v, sem_g).wait()
        pltpu.sync_copy(g_v, out_hbm.at[pl.ds(n0, ROWS)])


def kernel(x, index, supports_01):
    xp = x.transpose(1, 2, 0)
    tt = _pack_table(supports_01.T).reshape(VPAD * EMBED_DIM).reshape(
        VPAD, EMBED_DIM)
    idx = index.reshape(N).astype(jnp.int32)

    gather = functools.partial(
        pl.kernel,
        out_type=jax.ShapeDtypeStruct((N, EMBED_DIM), jnp.float32),
        mesh=plsc.VectorSubcoreMesh(core_axis_name="c", subcore_axis_name="s"),
        scratch_types=[
            pltpu.VMEM((ROWS,), jnp.int32),
            pltpu.VMEM((ROWS,), jnp.int32),
            pltpu.VMEM((ROWS, EMBED_DIM), jnp.float32),
            pltpu.SemaphoreType.DMA,
        ],
        compiler_params=pltpu.CompilerParams(use_tc_tiling_on_sc=False,
                                             needs_layout_passes=False),
    )(_sc_gather)

    g = gather(tt, idx)
    g128 = g.reshape(N * EMBED_DIM).reshape(N * EMBED_DIM // 128, 128)
    out_t = _mul_plane(xp, g128)
    return out_t.transpose(2, 0, 1)


# speedup vs baseline: 4.4212x; 1.1115x over previous
import functools

import jax
import jax.numpy as jnp
from jax import lax
from jax.experimental import pallas as pl
from jax.experimental.pallas import tpu as pltpu
from jax.experimental.pallas import tpu_sc as plsc

BATCH = 16384
HIST = 20
EMBED_DIM = 32
N = BATCH * HIST
VOCAB = 1000000
PACK = 128 // EMBED_DIM

NUM_CORES = 2
NUM_SUBCORES = 16
NUM_WORKERS = NUM_CORES * NUM_SUBCORES
BATCH_PER_WORKER = BATCH // NUM_WORKERS
CB = 64
ROWS = CB * HIST
NUM_CHUNKS = BATCH_PER_WORKER // CB
LANES = 16

VPAD = 1015808
TBLK = 16384
STRIP = TBLK // PACK


NBLK = 62
VTAIL = VOCAB - (NBLK - 1) * TBLK


def _tc_transpose_body(hbm_ref, tail_ref, out_ref, scr0, scr1, sem0, sem1):
    i = pl.program_id(0)

    def start(blk, scr, sem):
        @pl.when(blk < NBLK - 1)
        def _():
            pltpu.make_async_copy(
                hbm_ref.at[:, pl.ds(blk * TBLK, TBLK)], scr, sem).start()

        @pl.when(blk == NBLK - 1)
        def _():
            pltpu.make_async_copy(tail_ref, scr, sem).start()

    def wait(blk, scr, sem):
        @pl.when(blk < NBLK - 1)
        def _():
            pltpu.make_async_copy(
                hbm_ref.at[:, pl.ds(blk * TBLK, TBLK)], scr, sem).wait()

        @pl.when(blk == NBLK - 1)
        def _():
            pltpu.make_async_copy(tail_ref, scr, sem).wait()

    @pl.when(i == 0)
    def _():
        start(0, scr0, sem0)

    nxt = i + 1

    @pl.when(jnp.logical_and(nxt < NBLK, nxt % 2 == 0))
    def _():
        start(nxt, scr0, sem0)

    @pl.when(jnp.logical_and(nxt < NBLK, nxt % 2 == 1))
    def _():
        start(nxt, scr1, sem1)

    def emit(scr):
        t = scr[...]
        stacked = jnp.concatenate(
            [t[:, q * STRIP:(q + 1) * STRIP] for q in range(PACK)], axis=0)
        out_ref[...] = stacked.T

    @pl.when(i % 2 == 0)
    def _():
        wait(i, scr0, sem0)
        emit(scr0)

    @pl.when(i % 2 == 1)
    def _():
        wait(i, scr1, sem1)
        emit(scr1)


def _pack_table(table_t):
    tail = jnp.pad(
        lax.slice(table_t, (0, (NBLK - 1) * TBLK), (EMBED_DIM, VOCAB)),
        ((0, 0), (0, TBLK - VTAIL)))
    return pl.pallas_call(
        _tc_transpose_body,
        grid=(NBLK,),
        in_specs=[pl.BlockSpec(memory_space=pl.ANY),
                  pl.BlockSpec(memory_space=pl.ANY)],
        out_specs=pl.BlockSpec((STRIP, 128), lambda i: (i, 0)),
        out_shape=jax.ShapeDtypeStruct((VPAD // PACK, 128), jnp.float32),
        scratch_shapes=[
            pltpu.VMEM((EMBED_DIM, TBLK), jnp.float32),
            pltpu.VMEM((EMBED_DIM, TBLK), jnp.float32),
            pltpu.SemaphoreType.DMA,
            pltpu.SemaphoreType.DMA,
        ],
    )(table_t, tail)


SBLK = 2048
GROWS = HIST * EMBED_DIM // 128


def _tc_mul_body(x_ref, g_ref, o_ref):
    gv = g_ref[...]
    g3 = gv.reshape(SBLK, GROWS, 128)
    planes = []
    for k in range(GROWS):
        rowk = g3[:, k, :]
        for m in range(4):
            planes.append(rowk[:, 32 * m:32 * (m + 1)].T)
    gp = jnp.stack(planes, axis=0)
    xv = x_ref[...]
    o_ref[...] = gp / (1.0 + jnp.exp(-2.0 * xv))


def _mul_plane(xp, g128):
    return pl.pallas_call(
        _tc_mul_body,
        grid=(BATCH // SBLK,),
        in_specs=[
            pl.BlockSpec((HIST, EMBED_DIM, SBLK), lambda i: (0, 0, i)),
            pl.BlockSpec((GROWS * SBLK, 128), lambda i: (i, 0)),
        ],
        out_specs=pl.BlockSpec((HIST, EMBED_DIM, SBLK), lambda i: (0, 0, i)),
        out_shape=jax.ShapeDtypeStruct((HIST, EMBED_DIM, BATCH), jnp.float32),
        compiler_params=pltpu.CompilerParams(
            dimension_semantics=("parallel",)),
    )(xp, g128)


def _sc_gather(table_hbm, idx_hbm, out_hbm, idx_v, idx4_v, g_v, sem_g):
    wid = lax.axis_index("s") * NUM_CORES + lax.axis_index("c")
    base = wid * BATCH_PER_WORKER

    @pl.loop(0, NUM_CHUNKS)
    def _(ci):
        n0 = (base + ci * CB) * HIST
        pltpu.sync_copy(idx_hbm.at[pl.ds(n0, ROWS)], idx_v)

        @pl.loop(0, ROWS, step=LANES)
        def _(i):
            v = idx_v[pl.ds(i, LANES)]
            shr = jax.lax.shift_right_logical
            shl = jax.lax.shift_left
            idx4_v[pl.ds(i, LANES)] = (
                shl(shr(v, 14), 14) | shl(v & 4095, 2) | (shr(v, 12) & 3))

        pltpu.async_copy(table_hbm.at[idx4_v], g_v, sem_g).wait()
        pltpu.sync_copy(g_v, out_hbm.at[pl.ds(n0, ROWS)])


def kernel(x, index, supports_01):
    xp = x.transpose(1, 2, 0)
    tt = _pack_table(supports_01.T).reshape(VPAD * EMBED_DIM).reshape(
        VPAD, EMBED_DIM)
    idx = index.reshape(N).astype(jnp.int32)

    gather = functools.partial(
        pl.kernel,
        out_type=jax.ShapeDtypeStruct((N, EMBED_DIM), jnp.float32),
        mesh=plsc.VectorSubcoreMesh(core_axis_name="c", subcore_axis_name="s"),
        scratch_types=[
            pltpu.VMEM((ROWS,), jnp.int32),
            pltpu.VMEM((ROWS,), jnp.int32),
            pltpu.VMEM((ROWS, EMBED_DIM), jnp.float32),
            pltpu.SemaphoreType.DMA,
        ],
        compiler_params=pltpu.CompilerParams(use_tc_tiling_on_sc=False,
                                             needs_layout_passes=False),
    )(_sc_gather)

    g = gather(tt, idx)
    g128 = g.reshape(N * EMBED_DIM).reshape(N * EMBED_DIM // 128, 128)
    out_t = _mul_plane(xp, g128)
    return out_t.transpose(2, 0, 1)


# speedup vs baseline: 4.5201x; 1.0224x over previous
import functools

import jax
import jax.numpy as jnp
from jax import lax
from jax.experimental import pallas as pl
from jax.experimental.pallas import tpu as pltpu
from jax.experimental.pallas import tpu_sc as plsc

BATCH = 16384
HIST = 20
EMBED_DIM = 32
N = BATCH * HIST
VOCAB = 1000000
PACK = 128 // EMBED_DIM

NUM_CORES = 2
NUM_SUBCORES = 16
NUM_WORKERS = NUM_CORES * NUM_SUBCORES
BATCH_PER_WORKER = BATCH // NUM_WORKERS
CB = 128
ROWS = CB * HIST
NUM_CHUNKS = BATCH_PER_WORKER // CB
LANES = 16

VPAD = 1015808
TBLK = 16384
STRIP = TBLK // PACK


NBLK = 62
VTAIL = VOCAB - (NBLK - 1) * TBLK


def _tc_transpose_body(hbm_ref, tail_ref, out_ref, scr0, scr1, sem0, sem1):
    i = pl.program_id(0)

    def start(blk, scr, sem):
        @pl.when(blk < NBLK - 1)
        def _():
            pltpu.make_async_copy(
                hbm_ref.at[:, pl.ds(blk * TBLK, TBLK)], scr, sem).start()

        @pl.when(blk == NBLK - 1)
        def _():
            pltpu.make_async_copy(tail_ref, scr, sem).start()

    def wait(blk, scr, sem):
        @pl.when(blk < NBLK - 1)
        def _():
            pltpu.make_async_copy(
                hbm_ref.at[:, pl.ds(blk * TBLK, TBLK)], scr, sem).wait()

        @pl.when(blk == NBLK - 1)
        def _():
            pltpu.make_async_copy(tail_ref, scr, sem).wait()

    @pl.when(i == 0)
    def _():
        start(0, scr0, sem0)

    nxt = i + 1

    @pl.when(jnp.logical_and(nxt < NBLK, nxt % 2 == 0))
    def _():
        start(nxt, scr0, sem0)

    @pl.when(jnp.logical_and(nxt < NBLK, nxt % 2 == 1))
    def _():
        start(nxt, scr1, sem1)

    def emit(scr):
        t = scr[...]
        stacked = jnp.concatenate(
            [t[:, q * STRIP:(q + 1) * STRIP] for q in range(PACK)], axis=0)
        out_ref[...] = stacked.T

    @pl.when(i % 2 == 0)
    def _():
        wait(i, scr0, sem0)
        emit(scr0)

    @pl.when(i % 2 == 1)
    def _():
        wait(i, scr1, sem1)
        emit(scr1)


def _pack_table(table_t):
    tail = jnp.pad(
        lax.slice(table_t, (0, (NBLK - 1) * TBLK), (EMBED_DIM, VOCAB)),
        ((0, 0), (0, TBLK - VTAIL)))
    return pl.pallas_call(
        _tc_transpose_body,
        grid=(NBLK,),
        in_specs=[pl.BlockSpec(memory_space=pl.ANY),
                  pl.BlockSpec(memory_space=pl.ANY)],
        out_specs=pl.BlockSpec((STRIP, 128), lambda i: (i, 0)),
        out_shape=jax.ShapeDtypeStruct((VPAD // PACK, 128), jnp.float32),
        scratch_shapes=[
            pltpu.VMEM((EMBED_DIM, TBLK), jnp.float32),
            pltpu.VMEM((EMBED_DIM, TBLK), jnp.float32),
            pltpu.SemaphoreType.DMA,
            pltpu.SemaphoreType.DMA,
        ],
    )(table_t, tail)


SBLK = 2048
GROWS = HIST * EMBED_DIM // 128


def _tc_mul_body(x_ref, g_ref, o_ref):
    gv = g_ref[...]
    g3 = gv.reshape(SBLK, GROWS, 128)
    planes = []
    for k in range(GROWS):
        rowk = g3[:, k, :]
        for m in range(4):
            planes.append(rowk[:, 32 * m:32 * (m + 1)].T)
    gp = jnp.stack(planes, axis=0)
    xv = x_ref[...]
    o_ref[...] = gp / (1.0 + jnp.exp(-2.0 * xv))


def _mul_plane(xp, g128):
    return pl.pallas_call(
        _tc_mul_body,
        grid=(BATCH // SBLK,),
        in_specs=[
            pl.BlockSpec((HIST, EMBED_DIM, SBLK), lambda i: (0, 0, i)),
            pl.BlockSpec((GROWS * SBLK, 128), lambda i: (i, 0)),
        ],
        out_specs=pl.BlockSpec((HIST, EMBED_DIM, SBLK), lambda i: (0, 0, i)),
        out_shape=jax.ShapeDtypeStruct((HIST, EMBED_DIM, BATCH), jnp.float32),
        compiler_params=pltpu.CompilerParams(
            dimension_semantics=("parallel",)),
    )(xp, g128)


def _sc_gather(table_hbm, idx_hbm, out_hbm, idx_v, idx4_v, g_v, sem_g):
    wid = lax.axis_index("s") * NUM_CORES + lax.axis_index("c")
    base = wid * BATCH_PER_WORKER

    @pl.loop(0, NUM_CHUNKS)
    def _(ci):
        n0 = (base + ci * CB) * HIST
        pltpu.sync_copy(idx_hbm.at[pl.ds(n0, ROWS)], idx_v)

        @pl.loop(0, ROWS, step=LANES)
        def _(i):
            v = idx_v[pl.ds(i, LANES)]
            shr = jax.lax.shift_right_logical
            shl = jax.lax.shift_left
            idx4_v[pl.ds(i, LANES)] = (
                shl(shr(v, 14), 14) | shl(v & 4095, 2) | (shr(v, 12) & 3))

        pltpu.async_copy(table_hbm.at[idx4_v], g_v, sem_g).wait()
        pltpu.sync_copy(g_v, out_hbm.at[pl.ds(n0, ROWS)])


def kernel(x, index, supports_01):
    xp = x.transpose(1, 2, 0)
    tt = _pack_table(supports_01.T).reshape(VPAD * EMBED_DIM).reshape(
        VPAD, EMBED_DIM)
    idx = index.reshape(N).astype(jnp.int32)

    gather = functools.partial(
        pl.kernel,
        out_type=jax.ShapeDtypeStruct((N, EMBED_DIM), jnp.float32),
        mesh=plsc.VectorSubcoreMesh(core_axis_name="c", subcore_axis_name="s"),
        scratch_types=[
            pltpu.VMEM((ROWS,), jnp.int32),
            pltpu.VMEM((ROWS,), jnp.int32),
            pltpu.VMEM((ROWS, EMBED_DIM), jnp.float32),
            pltpu.SemaphoreType.DMA,
        ],
        compiler_params=pltpu.CompilerParams(use_tc_tiling_on_sc=False,
                                             needs_layout_passes=False),
    )(_sc_gather)

    g = gather(tt, idx)
    g128 = g.reshape(N * EMBED_DIM).reshape(N * EMBED_DIM // 128, 128)
    out_t = _mul_plane(xp, g128)
    return out_t.transpose(2, 0, 1)


# speedup vs baseline: 5.6488x; 1.2497x over previous
import functools

import jax
import jax.numpy as jnp
from jax import lax
from jax.experimental import pallas as pl
from jax.experimental.pallas import tpu as pltpu
from jax.experimental.pallas import tpu_sc as plsc

BATCH = 16384
HIST = 20
EMBED_DIM = 32
N = BATCH * HIST
VOCAB = 1000000
PACK = 128 // EMBED_DIM

NUM_CORES = 2
NUM_SUBCORES = 16
NUM_WORKERS = NUM_CORES * NUM_SUBCORES
BATCH_PER_WORKER = BATCH // NUM_WORKERS
CB = 128
ROWS = CB * HIST
NUM_CHUNKS = BATCH_PER_WORKER // CB
LANES = 16

VPAD = 1015808
TBLK = 16384
STRIP = TBLK // PACK


NBLK = 62
VTAIL = VOCAB - (NBLK - 1) * TBLK


def _tc_transpose_body(hbm_ref, tail_ref, out_ref, scr0, scr1, sem0, sem1):
    i = pl.program_id(0)

    def start(blk, scr, sem):
        @pl.when(blk < NBLK - 1)
        def _():
            pltpu.make_async_copy(
                hbm_ref.at[:, pl.ds(blk * TBLK, TBLK)], scr, sem).start()

        @pl.when(blk == NBLK - 1)
        def _():
            pltpu.make_async_copy(tail_ref, scr, sem).start()

    def wait(blk, scr, sem):
        @pl.when(blk < NBLK - 1)
        def _():
            pltpu.make_async_copy(
                hbm_ref.at[:, pl.ds(blk * TBLK, TBLK)], scr, sem).wait()

        @pl.when(blk == NBLK - 1)
        def _():
            pltpu.make_async_copy(tail_ref, scr, sem).wait()

    @pl.when(i == 0)
    def _():
        start(0, scr0, sem0)

    nxt = i + 1

    @pl.when(jnp.logical_and(nxt < NBLK, nxt % 2 == 0))
    def _():
        start(nxt, scr0, sem0)

    @pl.when(jnp.logical_and(nxt < NBLK, nxt % 2 == 1))
    def _():
        start(nxt, scr1, sem1)

    def emit(scr):
        t = scr[...]
        stacked = jnp.concatenate(
            [t[:, q * STRIP:(q + 1) * STRIP] for q in range(PACK)], axis=0)
        out_ref[...] = stacked.T

    @pl.when(i % 2 == 0)
    def _():
        wait(i, scr0, sem0)
        emit(scr0)

    @pl.when(i % 2 == 1)
    def _():
        wait(i, scr1, sem1)
        emit(scr1)


def _pack_table(table_t):
    tail = jnp.pad(
        lax.slice(table_t, (0, (NBLK - 1) * TBLK), (EMBED_DIM, VOCAB)),
        ((0, 0), (0, TBLK - VTAIL)))
    return pl.pallas_call(
        _tc_transpose_body,
        grid=(NBLK,),
        in_specs=[pl.BlockSpec(memory_space=pl.ANY),
                  pl.BlockSpec(memory_space=pl.ANY)],
        out_specs=pl.BlockSpec((STRIP, 128), lambda i: (i, 0)),
        out_shape=jax.ShapeDtypeStruct((VPAD // PACK, 128), jnp.float32),
        scratch_shapes=[
            pltpu.VMEM((EMBED_DIM, TBLK), jnp.float32),
            pltpu.VMEM((EMBED_DIM, TBLK), jnp.float32),
            pltpu.SemaphoreType.DMA,
            pltpu.SemaphoreType.DMA,
        ],
    )(table_t, tail)


SBLK = 2048
GROWS = HIST * EMBED_DIM // 128


def _tc_mul_body(x_ref, g_ref, o_ref):
    gv = g_ref[...]
    g3 = gv.reshape(SBLK, GROWS, 128)
    pieces = [
        g3[:, k, :].T.reshape(4, EMBED_DIM, SBLK)
        for k in range(GROWS)
    ]
    gp = jnp.concatenate(pieces, axis=0)
    xv = x_ref[...]
    o_ref[...] = gp / (1.0 + jnp.exp(-2.0 * xv))


def _mul_plane(xp, g128):
    return pl.pallas_call(
        _tc_mul_body,
        grid=(BATCH // SBLK,),
        in_specs=[
            pl.BlockSpec((HIST, EMBED_DIM, SBLK), lambda i: (0, 0, i)),
            pl.BlockSpec((GROWS * SBLK, 128), lambda i: (i, 0)),
        ],
        out_specs=pl.BlockSpec((HIST, EMBED_DIM, SBLK), lambda i: (0, 0, i)),
        out_shape=jax.ShapeDtypeStruct((HIST, EMBED_DIM, BATCH), jnp.float32),
        compiler_params=pltpu.CompilerParams(
            dimension_semantics=("parallel",)),
    )(xp, g128)


def _sc_gather(table_hbm, idx_hbm, out_hbm, idx_v, idx4_v, g_v, sem_g):
    wid = lax.axis_index("s") * NUM_CORES + lax.axis_index("c")
    base = wid * BATCH_PER_WORKER

    @pl.loop(0, NUM_CHUNKS)
    def _(ci):
        n0 = (base + ci * CB) * HIST
        pltpu.sync_copy(idx_hbm.at[pl.ds(n0, ROWS)], idx_v)

        @pl.loop(0, ROWS, step=LANES)
        def _(i):
            v = idx_v[pl.ds(i, LANES)]
            shr = jax.lax.shift_right_logical
            shl = jax.lax.shift_left
            idx4_v[pl.ds(i, LANES)] = (
                shl(shr(v, 14), 14) | shl(v & 4095, 2) | (shr(v, 12) & 3))

        pltpu.async_copy(table_hbm.at[idx4_v], g_v, sem_g).wait()
        pltpu.sync_copy(g_v, out_hbm.at[pl.ds(n0, ROWS)])


def kernel(x, index, supports_01):
    xp = x.transpose(1, 2, 0)
    tt = _pack_table(supports_01.T).reshape(VPAD * EMBED_DIM).reshape(
        VPAD, EMBED_DIM)
    idx = index.reshape(N).astype(jnp.int32)

    gather = functools.partial(
        pl.kernel,
        out_type=jax.ShapeDtypeStruct((N, EMBED_DIM), jnp.float32),
        mesh=plsc.VectorSubcoreMesh(core_axis_name="c", subcore_axis_name="s"),
        scratch_types=[
            pltpu.VMEM((ROWS,), jnp.int32),
            pltpu.VMEM((ROWS,), jnp.int32),
            pltpu.VMEM((ROWS, EMBED_DIM), jnp.float32),
            pltpu.SemaphoreType.DMA,
        ],
        compiler_params=pltpu.CompilerParams(use_tc_tiling_on_sc=False,
                                             needs_layout_passes=False),
    )(_sc_gather)

    g = gather(tt, idx)
    g128 = g.reshape(N * EMBED_DIM).reshape(N * EMBED_DIM // 128, 128)
    out_t = _mul_plane(xp, g128)
    return out_t.transpose(2, 0, 1)
